# baseline (speedup 1.0000x reference)
import jax
import jax.numpy as jnp
from jax import lax
from jax.experimental import pallas as pl
from jax.experimental.pallas import tpu as pltpu

_KG = 4


def _build_kernel(batch, seq_len, n, hd, n_grp):
    nh = n // 2
    blk = _KG * n_grp

    def _body(x_ref, a1_ref, a2_ref, lcpt_ref, lppt_ref, w1bd_ref, w2bd_ref,
              w1bc_ref, b1bc_ref, w2bc_ref, b2bc_ref, pinv_ref, out_ref,
              lx_ref, lxp_ref):
        xb = x_ref[...].astype(jnp.bfloat16)
        lx_ref[...] = jnp.dot(xb, a1_ref[...],
                              preferred_element_type=jnp.float32)
        lxp_ref[...] = jnp.dot(xb, a2_ref[...],
                               preferred_element_type=jnp.float32)

        lcpt = lcpt_ref[...]
        lppt = lppt_ref[...]
        w1bd = w1bd_ref[...]
        w2bd = w2bd_ref[...]
        w1bc = w1bc_ref[...]
        b1bc = b1bc_ref[...]
        w2bc = w2bc_ref[...]
        b2bc = b2bc_ref[...]

        def step(t, hs):
            new_hs = []
            for g in range(n_grp):
                lht = jnp.dot(hs[g].astype(jnp.bfloat16), lcpt,
                              preferred_element_type=jnp.float32)
                gpre_g = jnp.dot(w1bd, lht.astype(jnp.bfloat16),
                                 preferred_element_type=jnp.float32)
                xb1 = jnp.concatenate(
                    [w1bc * lx_ref[pl.ds((_KG * g + k) * seq_len + t, 1), :]
                     + b1bc for k in range(_KG)], axis=0)
                gt = 0.5 + 0.5 * jnp.tanh(0.5 * (xb1 + gpre_g))
                rs, us = [], []
                for k in range(_KG):
                    gk = gt[2 * hd * k:2 * hd * (k + 1)]
                    rs.append(jnp.concatenate(
                        [gk[:hd, :nh], gk[hd:, :nh]], axis=1))
                    us.append(jnp.concatenate(
                        [gk[:hd, nh:], gk[hd:, nh:]], axis=1))
                rt = jnp.concatenate(rs, axis=0)
                u = jnp.concatenate(us, axis=0)
                agg = jnp.dot((rt * hs[g]).astype(jnp.bfloat16), lppt,
                              preferred_element_type=jnp.float32)
                cpre = jnp.dot(w2bd, agg.astype(jnp.bfloat16),
                               preferred_element_type=jnp.float32)
                xb2 = jnp.concatenate(
                    [w2bc * lxp_ref[pl.ds((_KG * g + k) * seq_len + t, 1), :]
                     + b2bc for k in range(_KG)], axis=0)
                ct = jnp.tanh(xb2 + cpre)
                new_hs.append(ct + u * (hs[g] - ct))
            return tuple(new_hs)

        hs0 = tuple(jnp.zeros((_KG * hd, n), jnp.float32)
                    for _ in range(n_grp))
        hs = lax.fori_loop(0, seq_len, step, hs0, unroll=6)
        pinv = pinv_ref[...]
        for g in range(n_grp):
            z = jnp.dot(hs[g], pinv, preferred_element_type=jnp.float32)
            for k in range(_KG):
                out_ref[_KG * g + k, :, :] = z[hd * k:hd * (k + 1), :].T

    grid = (batch // blk,)
    kt = blk * seq_len
    return pl.pallas_call(
        _body,
        grid=grid,
        in_specs=[
            pl.BlockSpec((kt, n), lambda b: (b, 0)),
            pl.BlockSpec((n, n), lambda b: (0, 0)),
            pl.BlockSpec((n, n), lambda b: (0, 0)),
            pl.BlockSpec((n, n), lambda b: (0, 0)),
            pl.BlockSpec((n, n), lambda b: (0, 0)),
            pl.BlockSpec((_KG * 2 * hd, _KG * hd), lambda b: (0, 0)),
            pl.BlockSpec((_KG * hd, _KG * hd), lambda b: (0, 0)),
            pl.BlockSpec((2 * hd, n), lambda b: (0, 0)),
            pl.BlockSpec((2 * hd, n), lambda b: (0, 0)),
            pl.BlockSpec((hd, n), lambda b: (0, 0)),
            pl.BlockSpec((hd, n), lambda b: (0, 0)),
            pl.BlockSpec((n, n), lambda b: (0, 0)),
        ],
        out_specs=pl.BlockSpec((blk, n, hd), lambda b: (b, 0, 0)),
        out_shape=jax.ShapeDtypeStruct((batch, n, hd), jnp.float32),
        scratch_shapes=[
            pltpu.VMEM((kt, n), jnp.float32),
            pltpu.VMEM((kt, n), jnp.float32),
        ],
        compiler_params=pltpu.CompilerParams(
            dimension_semantics=("parallel",)),
    )


def kernel(inputs, laplacian, w1_in, w1_h, b1, w2_in, w2_h, b2):
    inputs = inputs.astype(jnp.float32)
    b, seq_len, n = inputs.shape
    hd = w1_h.shape[0]
    n_grp = 1
    for cand_grp in (16, 8, 4, 2):
        if b % (cand_grp * _KG) == 0:
            n_grp = cand_grp
            break

    lap = laplacian.astype(jnp.float32)
    perm = jnp.concatenate([jnp.arange(0, n, 2), jnp.arange(1, n, 2)])
    a1 = lap.T
    a2 = a1[:, perm]
    lcpt = a1[perm, :]
    lppt = a1[perm][:, perm]
    pinv = jax.nn.one_hot(perm, n, dtype=jnp.float32)

    w1t = w1_h.astype(jnp.float32).T
    w2t = w2_h.astype(jnp.float32).T
    w1bd = jax.scipy.linalg.block_diag(*([w1t] * _KG))
    w2bd = jax.scipy.linalg.block_diag(*([w2t] * _KG))
    w1bc = jnp.broadcast_to(w1_in.reshape(-1)[:, None], (2 * hd, n))
    b1bc = jnp.broadcast_to(b1.reshape(-1)[:, None], (2 * hd, n))
    w2bc = jnp.broadcast_to(w2_in.reshape(-1)[:, None], (hd, n))
    b2bc = jnp.broadcast_to(b2.reshape(-1)[:, None], (hd, n))

    x2 = inputs.reshape(b * seq_len, n)
    fused = _build_kernel(b, seq_len, n, hd, n_grp)
    return fused(x2,
                 a1.astype(jnp.bfloat16), a2.astype(jnp.bfloat16),
                 lcpt.astype(jnp.bfloat16), lppt.astype(jnp.bfloat16),
                 w1bd.astype(jnp.bfloat16), w2bd.astype(jnp.bfloat16),
                 w1bc, b1bc, w2bc, b2bc, pinv)


# speedup vs baseline: 1.9012x; 1.9012x over previous
import jax
import jax.numpy as jnp
from jax import lax
from jax.experimental import pallas as pl
from jax.experimental.pallas import tpu as pltpu

_KG = 4


def _build_kernel(batch, seq_len, n, hd, n_grp):
    nh = n // 2
    blk = _KG * n_grp

    def _body(x_ref, a1_ref, a2_ref, lcpt_ref, lppt_ref, w1bd_ref, w2bd_ref,
              w1bc_ref, b1bc_ref, w2bc_ref, b2bc_ref, pinv_ref, out_ref,
              lx_ref, lxp_ref):
        xb = x_ref[...].astype(jnp.bfloat16)
        lx_ref[...] = jnp.dot(xb, a1_ref[...],
                              preferred_element_type=jnp.float32)
        lxp_ref[...] = jnp.dot(xb, a2_ref[...],
                               preferred_element_type=jnp.float32)

        lcpt = lcpt_ref[...]
        lppt = lppt_ref[...]
        w1bd = w1bd_ref[...]
        w2bd = w2bd_ref[...]
        w1bc = w1bc_ref[...]
        b1bc = b1bc_ref[...]
        w2bc = w2bc_ref[...]
        b2bc = b2bc_ref[...]

        def step(t, hs):
            lhts = [jnp.dot(hs[g].astype(jnp.bfloat16), lcpt,
                            preferred_element_type=jnp.float32)
                    for g in range(n_grp)]
            gpre = [jnp.dot(w1bd, lhts[g].astype(jnp.bfloat16),
                            preferred_element_type=jnp.float32)
                    for g in range(n_grp)]
            new_hs = []
            for g in range(n_grp):
                xb1 = jnp.concatenate(
                    [w1bc * lx_ref[pl.ds((_KG * g + k) * seq_len + t, 1), :]
                     + b1bc for k in range(_KG)], axis=0)
                gt = 0.5 + 0.5 * jnp.tanh(0.5 * (xb1 + gpre[g]))
                rs, us = [], []
                for k in range(_KG):
                    gk = gt[2 * hd * k:2 * hd * (k + 1)]
                    rs.append(jnp.concatenate(
                        [gk[:hd, :nh], gk[hd:, :nh]], axis=1))
                    us.append(jnp.concatenate(
                        [gk[:hd, nh:], gk[hd:, nh:]], axis=1))
                rt = jnp.concatenate(rs, axis=0)
                u = jnp.concatenate(us, axis=0)
                agg = jnp.dot((rt * hs[g]).astype(jnp.bfloat16), lppt,
                              preferred_element_type=jnp.float32)
                cpre = jnp.dot(w2bd, agg.astype(jnp.bfloat16),
                               preferred_element_type=jnp.float32)
                xb2 = jnp.concatenate(
                    [w2bc * lxp_ref[pl.ds((_KG * g + k) * seq_len + t, 1), :]
                     + b2bc for k in range(_KG)], axis=0)
                ct = jnp.tanh(xb2 + cpre)
                new_hs.append(ct + u * (hs[g] - ct))
            return tuple(new_hs)

        hs0 = tuple(jnp.zeros((_KG * hd, n), jnp.float32)
                    for _ in range(n_grp))
        hs = lax.fori_loop(0, seq_len, step, hs0, unroll=4)
        pinv = pinv_ref[...]
        for g in range(n_grp):
            z = jnp.dot(hs[g], pinv, preferred_element_type=jnp.float32)
            for k in range(_KG):
                out_ref[_KG * g + k, :, :] = z[hd * k:hd * (k + 1), :].T

    grid = (batch // blk,)
    kt = blk * seq_len
    return pl.pallas_call(
        _body,
        grid=grid,
        in_specs=[
            pl.BlockSpec((kt, n), lambda b: (b, 0)),
            pl.BlockSpec((n, n), lambda b: (0, 0)),
            pl.BlockSpec((n, n), lambda b: (0, 0)),
            pl.BlockSpec((n, n), lambda b: (0, 0)),
            pl.BlockSpec((n, n), lambda b: (0, 0)),
            pl.BlockSpec((_KG * 2 * hd, _KG * hd), lambda b: (0, 0)),
            pl.BlockSpec((_KG * hd, _KG * hd), lambda b: (0, 0)),
            pl.BlockSpec((2 * hd, n), lambda b: (0, 0)),
            pl.BlockSpec((2 * hd, n), lambda b: (0, 0)),
            pl.BlockSpec((hd, n), lambda b: (0, 0)),
            pl.BlockSpec((hd, n), lambda b: (0, 0)),
            pl.BlockSpec((n, n), lambda b: (0, 0)),
        ],
        out_specs=pl.BlockSpec((blk, n, hd), lambda b: (b, 0, 0)),
        out_shape=jax.ShapeDtypeStruct((batch, n, hd), jnp.float32),
        scratch_shapes=[
            pltpu.VMEM((kt, n), jnp.float32),
            pltpu.VMEM((kt, n), jnp.float32),
        ],
        compiler_params=pltpu.CompilerParams(
            dimension_semantics=("parallel",)),
    )


def kernel(inputs, laplacian, w1_in, w1_h, b1, w2_in, w2_h, b2):
    inputs = inputs.astype(jnp.float32)
    b, seq_len, n = inputs.shape
    hd = w1_h.shape[0]
    n_grp = 1
    for cand_grp in (16, 8, 4, 2):
        if b % (cand_grp * _KG) == 0:
            n_grp = cand_grp
            break

    lap = laplacian.astype(jnp.float32)
    perm = jnp.concatenate([jnp.arange(0, n, 2), jnp.arange(1, n, 2)])
    a1 = lap.T
    a2 = a1[:, perm]
    lcpt = a1[perm, :]
    lppt = a1[perm][:, perm]
    pinv = jax.nn.one_hot(perm, n, dtype=jnp.float32)

    w1t = w1_h.astype(jnp.float32).T
    w2t = w2_h.astype(jnp.float32).T
    w1bd = jax.scipy.linalg.block_diag(*([w1t] * _KG))
    w2bd = jax.scipy.linalg.block_diag(*([w2t] * _KG))
    w1bc = jnp.broadcast_to(w1_in.reshape(-1)[:, None], (2 * hd, n))
    b1bc = jnp.broadcast_to(b1.reshape(-1)[:, None], (2 * hd, n))
    w2bc = jnp.broadcast_to(w2_in.reshape(-1)[:, None], (hd, n))
    b2bc = jnp.broadcast_to(b2.reshape(-1)[:, None], (hd, n))

    x2 = inputs.reshape(b * seq_len, n)
    fused = _build_kernel(b, seq_len, n, hd, n_grp)
    return fused(x2,
                 a1.astype(jnp.bfloat16), a2.astype(jnp.bfloat16),
                 lcpt.astype(jnp.bfloat16), lppt.astype(jnp.bfloat16),
                 w1bd.astype(jnp.bfloat16), w2bd.astype(jnp.bfloat16),
                 w1bc, b1bc, w2bc, b2bc, pinv)


# speedup vs baseline: 1.9085x; 1.0038x over previous
import jax
import jax.numpy as jnp
from jax import lax
from jax.experimental import pallas as pl
from jax.experimental.pallas import tpu as pltpu

_KG = 4


def _build_kernel(batch, seq_len, n, hd, n_grp):
    nh = n // 2
    blk = _KG * n_grp

    def _body(x_ref, a1_ref, a2_ref, lcpt_ref, lppt_ref, w1bd_ref, w2bd_ref,
              w1bc_ref, b1bc_ref, w2bc_ref, b2bc_ref, pinv_ref, out_ref,
              lx_ref, lxp_ref):
        xb = x_ref[...].astype(jnp.bfloat16)
        lx_ref[...] = jnp.dot(xb, a1_ref[...],
                              preferred_element_type=jnp.float32)
        lxp_ref[...] = jnp.dot(xb, a2_ref[...],
                               preferred_element_type=jnp.float32)

        lcpt = lcpt_ref[...]
        lppt = lppt_ref[...]
        w1bd = w1bd_ref[...]
        w2bd = w2bd_ref[...]
        w1bc = w1bc_ref[...]
        b1bc = b1bc_ref[...]
        w2bc = w2bc_ref[...]
        b2bc = b2bc_ref[...]

        def step(t, hs):
            lhts = [jnp.dot(hs[g].astype(jnp.bfloat16), lcpt,
                            preferred_element_type=jnp.float32)
                    for g in range(n_grp)]
            gpre = [jnp.dot(w1bd, lhts[g].astype(jnp.bfloat16),
                            preferred_element_type=jnp.float32)
                    for g in range(n_grp)]
            new_hs = []
            for g in range(n_grp):
                xb1 = jnp.concatenate(
                    [w1bc * lx_ref[pl.ds((_KG * g + k) * seq_len + t, 1), :]
                     + b1bc for k in range(_KG)], axis=0)
                gt = 0.5 + 0.5 * jnp.tanh(0.5 * (xb1 + gpre[g]))
                rs, us = [], []
                for k in range(_KG):
                    gk = gt[2 * hd * k:2 * hd * (k + 1)]
                    rs.append(jnp.concatenate(
                        [gk[:hd, :nh], gk[hd:, :nh]], axis=1))
                    us.append(jnp.concatenate(
                        [gk[:hd, nh:], gk[hd:, nh:]], axis=1))
                rt = jnp.concatenate(rs, axis=0)
                u = jnp.concatenate(us, axis=0)
                agg = jnp.dot((rt * hs[g]).astype(jnp.bfloat16), lppt,
                              preferred_element_type=jnp.float32)
                cpre = jnp.dot(w2bd, agg.astype(jnp.bfloat16),
                               preferred_element_type=jnp.float32)
                xb2 = jnp.concatenate(
                    [w2bc * lxp_ref[pl.ds((_KG * g + k) * seq_len + t, 1), :]
                     + b2bc for k in range(_KG)], axis=0)
                ct = jnp.tanh(xb2 + cpre)
                new_hs.append(ct + u * (hs[g] - ct))
            return tuple(new_hs)

        hs0 = tuple(jnp.zeros((_KG * hd, n), jnp.float32)
                    for _ in range(n_grp))
        hs = lax.fori_loop(0, seq_len, step, hs0, unroll=6)
        pinv = pinv_ref[...]
        for g in range(n_grp):
            z = jnp.dot(hs[g], pinv, preferred_element_type=jnp.float32)
            for k in range(_KG):
                out_ref[_KG * g + k, :, :] = z[hd * k:hd * (k + 1), :].T

    grid = (batch // blk,)
    kt = blk * seq_len
    return pl.pallas_call(
        _body,
        grid=grid,
        in_specs=[
            pl.BlockSpec((kt, n), lambda b: (b, 0)),
            pl.BlockSpec((n, n), lambda b: (0, 0)),
            pl.BlockSpec((n, n), lambda b: (0, 0)),
            pl.BlockSpec((n, n), lambda b: (0, 0)),
            pl.BlockSpec((n, n), lambda b: (0, 0)),
            pl.BlockSpec((_KG * 2 * hd, _KG * hd), lambda b: (0, 0)),
            pl.BlockSpec((_KG * hd, _KG * hd), lambda b: (0, 0)),
            pl.BlockSpec((2 * hd, n), lambda b: (0, 0)),
            pl.BlockSpec((2 * hd, n), lambda b: (0, 0)),
            pl.BlockSpec((hd, n), lambda b: (0, 0)),
            pl.BlockSpec((hd, n), lambda b: (0, 0)),
            pl.BlockSpec((n, n), lambda b: (0, 0)),
        ],
        out_specs=pl.BlockSpec((blk, n, hd), lambda b: (b, 0, 0)),
        out_shape=jax.ShapeDtypeStruct((batch, n, hd), jnp.float32),
        scratch_shapes=[
            pltpu.VMEM((kt, n), jnp.float32),
            pltpu.VMEM((kt, n), jnp.float32),
        ],
        compiler_params=pltpu.CompilerParams(
            dimension_semantics=("parallel",)),
    )


def kernel(inputs, laplacian, w1_in, w1_h, b1, w2_in, w2_h, b2):
    inputs = inputs.astype(jnp.float32)
    b, seq_len, n = inputs.shape
    hd = w1_h.shape[0]
    n_grp = 1
    for cand_grp in (8, 4, 2):
        if b % (cand_grp * _KG) == 0:
            n_grp = cand_grp
            break

    lap = laplacian.astype(jnp.float32)
    perm = jnp.concatenate([jnp.arange(0, n, 2), jnp.arange(1, n, 2)])
    a1 = lap.T
    a2 = a1[:, perm]
    lcpt = a1[perm, :]
    lppt = a1[perm][:, perm]
    pinv = jax.nn.one_hot(perm, n, dtype=jnp.float32)

    w1t = w1_h.astype(jnp.float32).T
    w2t = w2_h.astype(jnp.float32).T
    w1bd = jax.scipy.linalg.block_diag(*([w1t] * _KG))
    w2bd = jax.scipy.linalg.block_diag(*([w2t] * _KG))
    w1bc = jnp.broadcast_to(w1_in.reshape(-1)[:, None], (2 * hd, n))
    b1bc = jnp.broadcast_to(b1.reshape(-1)[:, None], (2 * hd, n))
    w2bc = jnp.broadcast_to(w2_in.reshape(-1)[:, None], (hd, n))
    b2bc = jnp.broadcast_to(b2.reshape(-1)[:, None], (hd, n))

    x2 = inputs.reshape(b * seq_len, n)
    fused = _build_kernel(b, seq_len, n, hd, n_grp)
    return fused(x2,
                 a1.astype(jnp.bfloat16), a2.astype(jnp.bfloat16),
                 lcpt.astype(jnp.bfloat16), lppt.astype(jnp.bfloat16),
                 w1bd.astype(jnp.bfloat16), w2bd.astype(jnp.bfloat16),
                 w1bc, b1bc, w2bc, b2bc, pinv)


# speedup vs baseline: 1.9358x; 1.0143x over previous
import jax
import jax.numpy as jnp
from jax import lax
from jax.experimental import pallas as pl
from jax.experimental.pallas import tpu as pltpu

_KG = 4


def _build_kernel(batch, seq_len, n, hd, n_grp):
    nh = n // 2
    blk = _KG * n_grp

    def _body(x_ref, a1_ref, a2_ref, lcpt_ref, lppt_ref, w1bd_ref, w2bd_ref,
              w1bc_ref, b1bc_ref, w2bc_ref, b2bc_ref, pinv_ref, out_ref,
              lx_ref, lxp_ref):
        xb = x_ref[...].astype(jnp.bfloat16)
        lx_ref[...] = jnp.dot(xb, a1_ref[...],
                              preferred_element_type=jnp.float32)
        lxp_ref[...] = jnp.dot(xb, a2_ref[...],
                               preferred_element_type=jnp.float32)

        lcpt = lcpt_ref[...]
        lppt = lppt_ref[...]
        w1bd = w1bd_ref[...]
        w2bd = w2bd_ref[...]
        w1bc = w1bc_ref[...]
        b1bc = b1bc_ref[...]
        w2bc = w2bc_ref[...]
        b2bc = b2bc_ref[...]

        def step(t, hs):
            lhts = [jnp.dot(hs[g].astype(jnp.bfloat16), lcpt,
                            preferred_element_type=jnp.float32)
                    for g in range(n_grp)]
            gpre = [jnp.dot(w1bd, lhts[g].astype(jnp.bfloat16),
                            preferred_element_type=jnp.float32)
                    for g in range(n_grp)]
            new_hs = []
            for g in range(n_grp):
                xb1 = jnp.concatenate(
                    [w1bc * lx_ref[pl.ds((_KG * g + k) * seq_len + t, 1), :]
                     + b1bc for k in range(_KG)], axis=0)
                gt = 0.5 + 0.5 * jnp.tanh(0.5 * (xb1 + gpre[g]))
                rs, us = [], []
                for k in range(_KG):
                    gk = gt[2 * hd * k:2 * hd * (k + 1)]
                    rs.append(jnp.concatenate(
                        [gk[:hd, :nh], gk[hd:, :nh]], axis=1))
                    us.append(jnp.concatenate(
                        [gk[:hd, nh:], gk[hd:, nh:]], axis=1))
                rt = jnp.concatenate(rs, axis=0)
                u = jnp.concatenate(us, axis=0)
                agg = jnp.dot((rt * hs[g]).astype(jnp.bfloat16), lppt,
                              preferred_element_type=jnp.float32)
                cpre = jnp.dot(w2bd, agg.astype(jnp.bfloat16),
                               preferred_element_type=jnp.float32)
                xb2 = jnp.concatenate(
                    [w2bc * lxp_ref[pl.ds((_KG * g + k) * seq_len + t, 1), :]
                     + b2bc for k in range(_KG)], axis=0)
                ct = jnp.tanh(xb2 + cpre)
                new_hs.append(ct + u * (hs[g] - ct))
            return tuple(new_hs)

        hs0 = tuple(jnp.zeros((_KG * hd, n), jnp.float32)
                    for _ in range(n_grp))
        hs = lax.fori_loop(0, seq_len, step, hs0, unroll=6)
        pinv = pinv_ref[...]
        for g in range(n_grp):
            z = jnp.dot(hs[g], pinv, preferred_element_type=jnp.float32)
            for k in range(_KG):
                out_ref[_KG * g + k, :, :] = z[hd * k:hd * (k + 1), :].T

    grid = (batch // blk,)
    kt = blk * seq_len
    return pl.pallas_call(
        _body,
        grid=grid,
        in_specs=[
            pl.BlockSpec((kt, n), lambda b: (b, 0)),
            pl.BlockSpec((n, n), lambda b: (0, 0)),
            pl.BlockSpec((n, n), lambda b: (0, 0)),
            pl.BlockSpec((n, n), lambda b: (0, 0)),
            pl.BlockSpec((n, n), lambda b: (0, 0)),
            pl.BlockSpec((_KG * 2 * hd, _KG * hd), lambda b: (0, 0)),
            pl.BlockSpec((_KG * hd, _KG * hd), lambda b: (0, 0)),
            pl.BlockSpec((2 * hd, n), lambda b: (0, 0)),
            pl.BlockSpec((2 * hd, n), lambda b: (0, 0)),
            pl.BlockSpec((hd, n), lambda b: (0, 0)),
            pl.BlockSpec((hd, n), lambda b: (0, 0)),
            pl.BlockSpec((n, n), lambda b: (0, 0)),
        ],
        out_specs=pl.BlockSpec((blk, n, hd), lambda b: (b, 0, 0)),
        out_shape=jax.ShapeDtypeStruct((batch, n, hd), jnp.float32),
        scratch_shapes=[
            pltpu.VMEM((kt, n), jnp.float32),
            pltpu.VMEM((kt, n), jnp.float32),
        ],
        compiler_params=pltpu.CompilerParams(
            dimension_semantics=("parallel",)),
    )


def kernel(inputs, laplacian, w1_in, w1_h, b1, w2_in, w2_h, b2):
    inputs = inputs.astype(jnp.float32)
    b, seq_len, n = inputs.shape
    hd = w1_h.shape[0]
    n_grp = 1
    for cand_grp in (16, 8, 4, 2):
        if b % (cand_grp * _KG) == 0:
            n_grp = cand_grp
            break

    lap = laplacian.astype(jnp.float32)
    perm = jnp.concatenate([jnp.arange(0, n, 2), jnp.arange(1, n, 2)])
    a1 = lap.T
    a2 = a1[:, perm]
    lcpt = a1[perm, :]
    lppt = a1[perm][:, perm]
    pinv = jax.nn.one_hot(perm, n, dtype=jnp.float32)

    w1t = w1_h.astype(jnp.float32).T
    w2t = w2_h.astype(jnp.float32).T
    w1bd = jax.scipy.linalg.block_diag(*([w1t] * _KG))
    w2bd = jax.scipy.linalg.block_diag(*([w2t] * _KG))
    w1bc = jnp.broadcast_to(w1_in.reshape(-1)[:, None], (2 * hd, n))
    b1bc = jnp.broadcast_to(b1.reshape(-1)[:, None], (2 * hd, n))
    w2bc = jnp.broadcast_to(w2_in.reshape(-1)[:, None], (hd, n))
    b2bc = jnp.broadcast_to(b2.reshape(-1)[:, None], (hd, n))

    x2 = inputs.reshape(b * seq_len, n)
    fused = _build_kernel(b, seq_len, n, hd, n_grp)
    return fused(x2,
                 a1.astype(jnp.bfloat16), a2.astype(jnp.bfloat16),
                 lcpt.astype(jnp.bfloat16), lppt.astype(jnp.bfloat16),
                 w1bd.astype(jnp.bfloat16), w2bd.astype(jnp.bfloat16),
                 w1bc, b1bc, w2bc, b2bc, pinv)
